# baseline (speedup 1.0000x reference)
import functools

import jax
import jax.numpy as jnp
from jax import lax
from jax.experimental import pallas as pl
from jax.experimental.pallas import tpu as pltpu
from jax.experimental.pallas import tpu_sc as plsc

_VOCAB = 1000000
_EMBED = 32
_NCLASS = 5
_DOCLEN = 200

_CHUNK = 8
_NTILES = 16
_NCHUNKS = _DOCLEN // _CHUNK


def _lane_shuffle(x, perm):
    return x.at[perm].get(mode="promise_in_bounds")


def _lane_sum(x):
    lanes = lax.iota(jnp.int32, 16)
    for sh in (8, 4, 2, 1):
        x = x + _lane_shuffle(x, lanes ^ sh)
    return x


def _lane_max(x):
    lanes = lax.iota(jnp.int32, 16)
    for sh in (8, 4, 2, 1):
        x = jnp.maximum(x, _lane_shuffle(x, lanes ^ sh))
    return x


def _sc_body(data_hbm, w3_hbm, b_hbm, table_hbm, out_hbm,
             idx_v, rows_v, w_v, part_v, all_v, bvec_v, out_v, shared, sem):
    cid = lax.axis_index("c")
    sid = lax.axis_index("s")

    @pl.when(cid == 0)
    def _core0():
        zero16f = jnp.zeros((16,), jnp.float32)
        has_b = sid < (_NCHUNKS - _NTILES)

        @pl.when(jnp.logical_not(has_b))
        def _():
            idx_v[...] = jnp.zeros((16,), jnp.int32)

        pltpu.sync_copy(data_hbm.at[pl.ds(sid * _CHUNK, _CHUNK)],
                        idx_v.at[pl.ds(0, _CHUNK)])

        @pl.when(has_b)
        def _():
            pltpu.sync_copy(
                data_hbm.at[pl.ds(_NTILES * _CHUNK + sid * _CHUNK, _CHUNK)],
                idx_v.at[pl.ds(_CHUNK, _CHUNK)])

        gather = pltpu.async_copy(table_hbm.at[idx_v], rows_v, sem)

        for c in range(_NCLASS):
            pltpu.sync_copy(w3_hbm.at[c, pl.ds(sid * _CHUNK, _CHUNK)],
                            w_v.at[c, pl.ds(0, _CHUNK)])

        @pl.when(has_b)
        def _():
            for c in range(_NCLASS):
                pltpu.sync_copy(
                    w3_hbm.at[c, pl.ds(_NTILES * _CHUNK + sid * _CHUNK, _CHUNK)],
                    w_v.at[c, pl.ds(_CHUNK, _CHUNK)])

        @pl.when(jnp.logical_not(has_b))
        def _():
            for c in range(_NCLASS):
                for j in range(_CHUNK, 2 * _CHUNK):
                    w_v[c, j, pl.ds(0, 16)] = zero16f
                    w_v[c, j, pl.ds(16, 16)] = zero16f

        gather.wait()

        acc = [zero16f for _ in range(_NCLASS)]
        for j in range(2 * _CHUNK):
            e0 = rows_v[j, pl.ds(0, 16)]
            e1 = rows_v[j, pl.ds(16, 16)]
            for c in range(_NCLASS):
                acc[c] = acc[c] + e0 * w_v[c, j, pl.ds(0, 16)]
                acc[c] = acc[c] + e1 * w_v[c, j, pl.ds(16, 16)]
        for c in range(_NCLASS):
            part_v[c, pl.ds(0, 16)] = acc[c]

        pltpu.sync_copy(part_v, shared.at[sid])
        plsc.subcore_barrier()

        @pl.when(sid == 0)
        def _():
            pltpu.sync_copy(shared, all_v)
            pltpu.sync_copy(b_hbm, bvec_v)
            lanes = lax.iota(jnp.int32, 16)
            logits = bvec_v[...]
            for c in range(_NCLASS):
                tot = all_v[0, c, pl.ds(0, 16)]
                for t in range(1, _NTILES):
                    tot = tot + all_v[t, c, pl.ds(0, 16)]
                logits = logits + jnp.where(lanes == c, _lane_sum(tot), 0.0)
            logits = jnp.where(lanes < _NCLASS, logits, jnp.float32(-1e30))
            e = jnp.exp(logits - _lane_max(logits))
            e = jnp.where(lanes < _NCLASS, e, jnp.float32(0.0))
            out_v[...] = e / _lane_sum(e)
            pltpu.sync_copy(out_v, out_hbm)


_sc_kernel = functools.partial(
    pl.kernel,
    mesh=plsc.VectorSubcoreMesh(core_axis_name="c", subcore_axis_name="s"),
    out_type=jax.ShapeDtypeStruct((16,), jnp.float32),
    compiler_params=pltpu.CompilerParams(use_tc_tiling_on_sc=False),
    scratch_types=[
        pltpu.VMEM((16,), jnp.int32),
        pltpu.VMEM((2 * _CHUNK, _EMBED), jnp.float32),
        pltpu.VMEM((_NCLASS, 2 * _CHUNK, _EMBED), jnp.float32),
        pltpu.VMEM((_NCLASS, 16), jnp.float32),
        pltpu.VMEM((_NTILES, _NCLASS, 16), jnp.float32),
        pltpu.VMEM((16,), jnp.float32),
        pltpu.VMEM((16,), jnp.float32),
        pltpu.VMEM_SHARED((_NTILES, _NCLASS, 16), jnp.float32),
        pltpu.SemaphoreType.DMA,
    ],
)(_sc_body)


@jax.jit
def kernel(data, embed_table, W, b):
    data_i = data.astype(jnp.int32)
    w3 = W.reshape(_NCLASS, _DOCLEN, _EMBED)
    b16 = jnp.pad(b.astype(jnp.float32), (0, 16 - _NCLASS))
    out16 = _sc_kernel(data_i, w3, b16, embed_table)
    return out16[:_NCLASS][None, :]


# speedup vs baseline: 22.6947x; 22.6947x over previous
import jax
import jax.numpy as jnp
from jax.experimental import pallas as pl
from jax.experimental.pallas import tpu as pltpu

_VOCAB = 1000000
_EMBED = 32
_NCLASS = 5
_DOCLEN = 200
_NBUF = 8


def _body(data_sm, tabT_hbm, w_ref, b_ref, out_ref, blk_ref, sem):
    def _copy(t):
        v = data_sm[t]
        tc = pl.multiple_of((v // 128) * 128, 128)
        return pltpu.make_async_copy(
            tabT_hbm.at[:, pl.ds(tc, 128)],
            blk_ref.at[pl.ds(_EMBED * t, _EMBED), :],
            sem.at[t % _NBUF],
        )

    for t in range(_NBUF):
        _copy(t).start()
    for t in range(_NBUF, _DOCLEN):
        _copy(t).start()
        _copy(t - _NBUF).wait()
    for t in range(_DOCLEN - _NBUF, _DOCLEN):
        _copy(t).wait()

    lane = jax.lax.broadcasted_iota(jnp.int32, (1, 128), 1)
    cols = []
    for t in range(_DOCLEN):
        vm = data_sm[t] % 128
        blk = blk_ref[_EMBED * t:_EMBED * (t + 1), :]
        mask = (lane == vm).astype(jnp.float32)
        cols.append(jnp.sum(blk * mask, axis=1, keepdims=True))
    emb = jnp.concatenate(cols, axis=1)
    embT = emb.T

    acc = jnp.zeros((_NCLASS, _EMBED), jnp.float32)
    for t in range(_DOCLEN):
        e_t = embT[t:t + 1, :]
        w_t = w_ref[:, pl.ds(t * _EMBED, _EMBED)]
        acc = acc + e_t * w_t
    logits = jnp.sum(acc, axis=1, keepdims=True).T + b_ref[...]
    m = jnp.max(logits, axis=1, keepdims=True)
    e = jnp.exp(logits - m)
    out_ref[...] = e / jnp.sum(e, axis=1, keepdims=True)


_tc_kernel = pl.pallas_call(
    _body,
    grid_spec=pltpu.PrefetchScalarGridSpec(
        num_scalar_prefetch=1,
        grid=(1,),
        in_specs=[
            pl.BlockSpec(memory_space=pl.ANY),
            pl.BlockSpec((_NCLASS, _EMBED * _DOCLEN), lambda i, *_: (0, 0)),
            pl.BlockSpec((1, _NCLASS), lambda i, *_: (0, 0)),
        ],
        out_specs=pl.BlockSpec((1, _NCLASS), lambda i, *_: (0, 0)),
        scratch_shapes=[
            pltpu.VMEM((_EMBED * _DOCLEN, 128), jnp.float32),
            pltpu.SemaphoreType.DMA((_NBUF,)),
        ],
    ),
    out_shape=jax.ShapeDtypeStruct((1, _NCLASS), jnp.float32),
)


@jax.jit
def kernel(data, embed_table, W, b):
    data_i = data.astype(jnp.int32)
    tabT = embed_table.T
    return _tc_kernel(data_i, tabT, W, b.reshape(1, _NCLASS))


# speedup vs baseline: 25.3652x; 1.1177x over previous
import jax
import jax.numpy as jnp
from jax.experimental import pallas as pl
from jax.experimental.pallas import tpu as pltpu

_VOCAB = 1000000
_EMBED = 32
_NCLASS = 5
_DOCLEN = 200
_NBUF = 32


def _body(data_sm, tabT_hbm, w_ref, b_ref, out_ref, blk_ref, sem):
    def _copy(t):
        v = data_sm[t]
        tc = pl.multiple_of((v // 128) * 128, 128)
        return pltpu.make_async_copy(
            tabT_hbm.at[:, pl.ds(tc, 128)],
            blk_ref.at[pl.ds(_EMBED * t, _EMBED), :],
            sem.at[t % _NBUF],
        )

    lane = jax.lax.broadcasted_iota(jnp.int32, (1, 128), 1)
    cols = []

    def _extract(t):
        vm = data_sm[t] % 128
        blk = blk_ref[_EMBED * t:_EMBED * (t + 1), :]
        mask = (lane == vm).astype(jnp.float32)
        cols.append(jnp.sum(blk * mask, axis=1, keepdims=True))

    for t in range(_NBUF):
        _copy(t).start()
    for t in range(_NBUF, _DOCLEN):
        _copy(t).start()
        _copy(t - _NBUF).wait()
        _extract(t - _NBUF)
    for t in range(_DOCLEN - _NBUF, _DOCLEN):
        _copy(t).wait()
        _extract(t)

    emb = jnp.concatenate(cols, axis=1)
    embT = emb.T
    acc = jnp.zeros((_NCLASS, _EMBED), jnp.float32)
    for t in range(_DOCLEN):
        e_t = embT[t:t + 1, :]
        w_t = w_ref[:, pl.ds(t * _EMBED, _EMBED)]
        acc = acc + e_t * w_t
    logits = jnp.sum(acc, axis=1, keepdims=True).T + b_ref[...]
    m = jnp.max(logits, axis=1, keepdims=True)
    e = jnp.exp(logits - m)
    out_ref[...] = e / jnp.sum(e, axis=1, keepdims=True)


_tc_kernel = pl.pallas_call(
    _body,
    grid_spec=pltpu.PrefetchScalarGridSpec(
        num_scalar_prefetch=1,
        grid=(1,),
        in_specs=[
            pl.BlockSpec(memory_space=pl.ANY),
            pl.BlockSpec((_NCLASS, _EMBED * _DOCLEN), lambda i, *_: (0, 0)),
            pl.BlockSpec((1, _NCLASS), lambda i, *_: (0, 0)),
        ],
        out_specs=pl.BlockSpec((1, _NCLASS), lambda i, *_: (0, 0)),
        scratch_shapes=[
            pltpu.VMEM((_EMBED * _DOCLEN, 128), jnp.float32),
            pltpu.SemaphoreType.DMA((_NBUF,)),
        ],
    ),
    out_shape=jax.ShapeDtypeStruct((1, _NCLASS), jnp.float32),
)


@jax.jit
def kernel(data, embed_table, W, b):
    data_i = data.astype(jnp.int32)
    tabT = embed_table.T
    return _tc_kernel(data_i, tabT, W, b.reshape(1, _NCLASS))
